# baseline (speedup 1.0000x reference)
import functools

import jax
import jax.numpy as jnp
from jax import lax
from jax.experimental import pallas as pl
from jax.experimental.pallas import tpu as pltpu
from jax.experimental.pallas import tpu_sc as plsc

B = 16384
DEG = 50
ED = 32
NV = 1000000
NC, NS, L = 2, 16, 16
NW = NC * NS
ROWS_PER_W = B // NW
CHUNK = 32
NCHUNK = ROWS_PER_W // CHUNK


def _sc_side_body(nodes, neibs3, table, out, sidx, nidx, nbuf, sbuf, obuf,
                  sem, sem_s):
    wid = lax.axis_index("s") * NC + lax.axis_index("c")
    inv = jnp.float32(1.0 / DEG)

    def chunk_body(c, _):
        blk = wid * NCHUNK + c
        row0 = blk * CHUNK
        pltpu.sync_copy(nodes.at[pl.ds(row0, CHUNK)], sidx)
        pltpu.sync_copy(neibs3.at[blk], nidx)
        pltpu.make_async_copy(table.at[sidx], sbuf, sem_s).start()

        def fire(r, _):
            pltpu.make_async_copy(table.at[nidx.at[r]], nbuf.at[r], sem).start()
            return 0
        lax.fori_loop(0, CHUNK, fire, 0)

        def drain(r, _):
            pltpu.make_async_copy(table.at[nidx.at[r]], nbuf.at[r], sem).wait()
            return 0
        lax.fori_loop(0, CHUNK, drain, 0)
        pltpu.make_async_copy(table.at[sidx], sbuf, sem_s).wait()

        def row(r, _):
            a00 = nbuf[r, 0, pl.ds(0, L)]
            a10 = nbuf[r, 0, pl.ds(L, L)]
            a01 = nbuf[r, 1, pl.ds(0, L)]
            a11 = nbuf[r, 1, pl.ds(L, L)]
            for n in range(2, DEG, 2):
                a00 = a00 + nbuf[r, n, pl.ds(0, L)]
                a10 = a10 + nbuf[r, n, pl.ds(L, L)]
                a01 = a01 + nbuf[r, n + 1, pl.ds(0, L)]
                a11 = a11 + nbuf[r, n + 1, pl.ds(L, L)]
            obuf[r, pl.ds(0, L)] = sbuf[r, pl.ds(0, L)]
            obuf[r, pl.ds(L, L)] = sbuf[r, pl.ds(L, L)]
            obuf[r, pl.ds(2 * L, L)] = (a00 + a01) * inv
            obuf[r, pl.ds(3 * L, L)] = (a10 + a11) * inv
            return 0
        lax.fori_loop(0, CHUNK, row, 0)

        pltpu.sync_copy(obuf, out.at[pl.ds(row0, CHUNK)])
        return 0

    lax.fori_loop(0, NCHUNK, chunk_body, 0)


@functools.lru_cache(maxsize=1)
def _make_gather_mean():
    return pl.kernel(
        _sc_side_body,
        out_type=jax.ShapeDtypeStruct((B, 2 * ED), jnp.float32),
        mesh=plsc.VectorSubcoreMesh(core_axis_name="c", subcore_axis_name="s",
                                    num_cores=NC, num_subcores=NS),
        scratch_types=[
            pltpu.VMEM((CHUNK,), jnp.int32),
            pltpu.VMEM((CHUNK, DEG), jnp.int32),
            pltpu.VMEM((CHUNK, DEG, ED), jnp.float32),
            pltpu.VMEM((CHUNK, ED), jnp.float32),
            pltpu.VMEM((CHUNK, 2 * ED), jnp.float32),
            pltpu.SemaphoreType.DMA,
            pltpu.SemaphoreType.DMA,
        ],
        compiler_params=pltpu.CompilerParams(use_tc_tiling_on_sc=False),
    )


TW = 4096
NT = (NV + TW - 1) // TW
NVP = NT * TW


def _tr_body(src_ref, dst_ref):
    d_iota = lax.broadcasted_iota(jnp.int32, (ED, 128), 0)
    l_iota = lax.broadcasted_iota(jnp.int32, (ED, 128), 1)

    def compute(tail):
        ntail = NV - (NT - 1) * TW
        acc = None
        for m in range(4):
            if tail and (TW // 4) * m >= ntail:
                continue
            xc = src_ref[:, (TW // 4) * m:(TW // 4) * (m + 1)]
            if tail:
                col = lax.broadcasted_iota(jnp.int32, (ED, TW // 4), 1)
                xc = jnp.where(col + (TW // 4) * m < ntail, xc, 0.0)
            pm = (l_iota == d_iota + 32 * m).astype(jnp.float32)
            t = lax.dot_general(xc, pm, (((0,), (0,)), ((), ())),
                                preferred_element_type=jnp.float32)
            acc = t if acc is None else acc + t
        return acc

    i = pl.program_id(0)

    @pl.when(i != NT - 1)
    def _():
        dst_ref[...] = compute(False)

    @pl.when(i == NT - 1)
    def _():
        dst_ref[...] = compute(True)


def _linearize_table(t):
    out = pl.pallas_call(
        _tr_body,
        grid=(NT,),
        in_specs=[pl.BlockSpec((ED, TW), lambda i: (0, i))],
        out_specs=pl.BlockSpec((TW // 4, 128), lambda i: (i, 0)),
        out_shape=jax.ShapeDtypeStruct((NVP * ED // 128, 128), jnp.float32),
    )(t.T)
    return out.reshape(NVP, ED)


def _remap_idx(r):
    return (r & ~(TW - 1)) | ((r & (TW // 4 - 1)) << 2) | ((r >> 10) & 3)


RB = 2048
NB = B // RB


def _tc_body(xu_ref, xv_ref, wut_ref, bu_ref, wvt_ref, bv_ref, out_ref):
    u = jnp.dot(xu_ref[...], wut_ref[...],
                preferred_element_type=jnp.float32) + bu_ref[...]
    v = jnp.dot(xv_ref[...], wvt_ref[...],
                preferred_element_type=jnp.float32) + bv_ref[...]
    out_ref[...] = jnp.sum(u * v, axis=1)[:, None]


def _project_dot(xu, xv, wut, bu2, wvt, bv2):
    out = pl.pallas_call(
        _tc_body,
        grid=(NB,),
        in_specs=[
            pl.BlockSpec((RB, 2 * ED), lambda i: (i, 0)),
            pl.BlockSpec((RB, 2 * ED), lambda i: (i, 0)),
            pl.BlockSpec((2 * ED, ED), lambda i: (0, 0)),
            pl.BlockSpec((1, ED), lambda i: (0, 0)),
            pl.BlockSpec((2 * ED, ED), lambda i: (0, 0)),
            pl.BlockSpec((1, ED), lambda i: (0, 0)),
        ],
        out_specs=pl.BlockSpec((RB, 1), lambda i: (i, 0)),
        out_shape=jax.ShapeDtypeStruct((B, 1), jnp.float32),
    )(xu, xv, wut, bu2, wvt, bv2)
    return out.reshape(B)


def kernel(nodes_u, nodes_v, u_neibs, v_neibs, all_sels, u2e, v2e, Wu, bu, Wv, bv):
    del all_sels
    nodes_u = _remap_idx(nodes_u.astype(jnp.int32))
    nodes_v = _remap_idx(nodes_v.astype(jnp.int32))
    un3 = _remap_idx(u_neibs.astype(jnp.int32)).reshape(NW * NCHUNK, CHUNK, DEG)
    vn3 = _remap_idx(v_neibs.astype(jnp.int32)).reshape(NW * NCHUNK, CHUNK, DEG)
    u2e_lin = _linearize_table(u2e)
    v2e_lin = _linearize_table(v2e)
    gm = _make_gather_mean()
    xu = gm(nodes_u, un3, u2e_lin)
    xv = gm(nodes_v, vn3, v2e_lin)
    return _project_dot(xu, xv, Wu.T, bu[None, :], Wv.T, bv[None, :])


# speedup vs baseline: 1.2382x; 1.2382x over previous
import functools

import jax
import jax.numpy as jnp
from jax import lax
from jax.experimental import pallas as pl
from jax.experimental.pallas import tpu as pltpu
from jax.experimental.pallas import tpu_sc as plsc

B = 16384
DEG = 50
ED = 32
NV = 1000000
NC, NS, L = 2, 16, 16
NW = NC * NS
ROWS_PER_W = B // NW
CHUNK = 32
NCHUNK = ROWS_PER_W // CHUNK


def _sc_side_body(nodes, neibs3, table, out, sidx, nidx, nbuf, sbuf, obuf,
                  sem, sem_s):
    wid = lax.axis_index("s") * NC + lax.axis_index("c")
    inv = jnp.float32(1.0 / DEG)

    def chunk_body(c, _):
        blk = wid * NCHUNK + c
        row0 = blk * CHUNK
        pltpu.sync_copy(nodes.at[pl.ds(row0, CHUNK)], sidx)
        pltpu.sync_copy(neibs3.at[blk], nidx)
        pltpu.make_async_copy(table.at[sidx], sbuf, sem_s).start()

        def fire(r, _):
            pltpu.make_async_copy(table.at[nidx.at[r]], nbuf.at[r], sem).start()
            return 0
        lax.fori_loop(0, CHUNK, fire, 0)

        def drain(r, _):
            pltpu.make_async_copy(table.at[nidx.at[r]], nbuf.at[r], sem).wait()
            return 0
        lax.fori_loop(0, CHUNK, drain, 0)
        pltpu.make_async_copy(table.at[sidx], sbuf, sem_s).wait()

        def row(r, _):
            a00 = nbuf[r, 0, pl.ds(0, L)]
            a10 = nbuf[r, 0, pl.ds(L, L)]
            a01 = nbuf[r, 1, pl.ds(0, L)]
            a11 = nbuf[r, 1, pl.ds(L, L)]
            for n in range(2, DEG, 2):
                a00 = a00 + nbuf[r, n, pl.ds(0, L)]
                a10 = a10 + nbuf[r, n, pl.ds(L, L)]
                a01 = a01 + nbuf[r, n + 1, pl.ds(0, L)]
                a11 = a11 + nbuf[r, n + 1, pl.ds(L, L)]
            obuf[r, pl.ds(0, L)] = sbuf[r, pl.ds(0, L)]
            obuf[r, pl.ds(L, L)] = sbuf[r, pl.ds(L, L)]
            obuf[r, pl.ds(2 * L, L)] = (a00 + a01) * inv
            obuf[r, pl.ds(3 * L, L)] = (a10 + a11) * inv
            return 0
        lax.fori_loop(0, CHUNK, row, 0)

        pltpu.sync_copy(obuf, out.at[pl.ds(row0, CHUNK)])
        return 0

    lax.fori_loop(0, NCHUNK, chunk_body, 0)


@functools.lru_cache(maxsize=1)
def _make_gather_mean():
    return pl.kernel(
        _sc_side_body,
        out_type=jax.ShapeDtypeStruct((B, 2 * ED), jnp.float32),
        mesh=plsc.VectorSubcoreMesh(core_axis_name="c", subcore_axis_name="s",
                                    num_cores=NC, num_subcores=NS),
        scratch_types=[
            pltpu.VMEM((CHUNK,), jnp.int32),
            pltpu.VMEM((CHUNK, DEG), jnp.int32),
            pltpu.VMEM((CHUNK, DEG, ED), jnp.float32),
            pltpu.VMEM((CHUNK, ED), jnp.float32),
            pltpu.VMEM((CHUNK, 2 * ED), jnp.float32),
            pltpu.SemaphoreType.DMA,
            pltpu.SemaphoreType.DMA,
        ],
        compiler_params=pltpu.CompilerParams(use_tc_tiling_on_sc=False),
    )


TW = 8192
NT = (NV + TW - 1) // TW
NVP = NT * TW
SH = (TW // 4).bit_length() - 1


def _tr_body(src_ref, dst_ref):
    d_iota = lax.broadcasted_iota(jnp.int32, (ED, 128), 0)
    l_iota = lax.broadcasted_iota(jnp.int32, (ED, 128), 1)

    def compute(tail):
        ntail = NV - (NT - 1) * TW
        acc = None
        for m in range(4):
            if tail and (TW // 4) * m >= ntail:
                continue
            xc = src_ref[:, (TW // 4) * m:(TW // 4) * (m + 1)]
            if tail:
                col = lax.broadcasted_iota(jnp.int32, (ED, TW // 4), 1)
                xc = jnp.where(col + (TW // 4) * m < ntail, xc, 0.0)
            pm = (l_iota == d_iota + 32 * m).astype(jnp.float32)
            t = lax.dot_general(xc, pm, (((0,), (0,)), ((), ())),
                                preferred_element_type=jnp.float32)
            acc = t if acc is None else acc + t
        return acc

    i = pl.program_id(0)

    @pl.when(i != NT - 1)
    def _():
        dst_ref[...] = compute(False)

    @pl.when(i == NT - 1)
    def _():
        dst_ref[...] = compute(True)


def _linearize_table(t):
    out = pl.pallas_call(
        _tr_body,
        grid=(NT,),
        in_specs=[pl.BlockSpec((ED, TW), lambda i: (0, i))],
        out_specs=pl.BlockSpec((TW // 4, 128), lambda i: (i, 0)),
        out_shape=jax.ShapeDtypeStruct((NVP * ED // 128, 128), jnp.float32),
        compiler_params=pltpu.CompilerParams(fuse_transposed_lhs_in_matmul=True),
    )(t.T)
    return out.reshape(NVP, ED)


def _remap_idx(r):
    return (r & ~(TW - 1)) | ((r & (TW // 4 - 1)) << 2) | ((r >> SH) & 3)


RB = 2048
NB = B // RB


def _tc_body(xu_ref, xv_ref, wut_ref, bu_ref, wvt_ref, bv_ref, out_ref):
    u = jnp.dot(xu_ref[...], wut_ref[...],
                preferred_element_type=jnp.float32) + bu_ref[...]
    v = jnp.dot(xv_ref[...], wvt_ref[...],
                preferred_element_type=jnp.float32) + bv_ref[...]
    out_ref[...] = jnp.sum(u * v, axis=1)[:, None]


def _project_dot(xu, xv, wut, bu2, wvt, bv2):
    out = pl.pallas_call(
        _tc_body,
        grid=(NB,),
        in_specs=[
            pl.BlockSpec((RB, 2 * ED), lambda i: (i, 0)),
            pl.BlockSpec((RB, 2 * ED), lambda i: (i, 0)),
            pl.BlockSpec((2 * ED, ED), lambda i: (0, 0)),
            pl.BlockSpec((1, ED), lambda i: (0, 0)),
            pl.BlockSpec((2 * ED, ED), lambda i: (0, 0)),
            pl.BlockSpec((1, ED), lambda i: (0, 0)),
        ],
        out_specs=pl.BlockSpec((RB, 1), lambda i: (i, 0)),
        out_shape=jax.ShapeDtypeStruct((B, 1), jnp.float32),
    )(xu, xv, wut, bu2, wvt, bv2)
    return out.reshape(B)


def kernel(nodes_u, nodes_v, u_neibs, v_neibs, all_sels, u2e, v2e, Wu, bu, Wv, bv):
    del all_sels
    nodes_u = _remap_idx(nodes_u.astype(jnp.int32))
    nodes_v = _remap_idx(nodes_v.astype(jnp.int32))
    un3 = _remap_idx(u_neibs.astype(jnp.int32)).reshape(NW * NCHUNK, CHUNK, DEG)
    vn3 = _remap_idx(v_neibs.astype(jnp.int32)).reshape(NW * NCHUNK, CHUNK, DEG)
    u2e_lin = _linearize_table(u2e)
    v2e_lin = _linearize_table(v2e)
    gm = _make_gather_mean()
    xu = gm(nodes_u, un3, u2e_lin)
    xv = gm(nodes_v, vn3, v2e_lin)
    return _project_dot(xu, xv, Wu.T, bu[None, :], Wv.T, bv[None, :])


# speedup vs baseline: 1.4124x; 1.1407x over previous
import functools

import jax
import jax.numpy as jnp
from jax import lax
from jax.experimental import pallas as pl
from jax.experimental.pallas import tpu as pltpu
from jax.experimental.pallas import tpu_sc as plsc

B = 16384
DEG = 50
ED = 32
NV = 1000000
NC, NS, L = 2, 16, 16
NW = NC * NS
ROWS_PER_W = B // NW
CHUNK = 32
NCHUNK = ROWS_PER_W // CHUNK


def _sc_side_body(nodes, neibs3, table, out, sidx, nidx, nbuf, sbuf, obuf,
                  sem, sem_s):
    wid = lax.axis_index("s") * NC + lax.axis_index("c")
    inv = jnp.float32(1.0 / DEG)

    def chunk_body(c, _):
        blk = wid * NCHUNK + c
        row0 = blk * CHUNK
        pltpu.sync_copy(nodes.at[pl.ds(row0, CHUNK)], sidx)
        pltpu.sync_copy(neibs3.at[blk], nidx)
        pltpu.make_async_copy(table.at[sidx], sbuf, sem_s).start()

        def fire(r, _):
            pltpu.make_async_copy(table.at[nidx.at[r]], nbuf.at[r], sem).start()
            return 0
        lax.fori_loop(0, CHUNK, fire, 0)

        def drain(r, _):
            pltpu.make_async_copy(table.at[nidx.at[r]], nbuf.at[r], sem).wait()
            return 0
        lax.fori_loop(0, CHUNK, drain, 0)
        pltpu.make_async_copy(table.at[sidx], sbuf, sem_s).wait()

        def row(r, _):
            a00 = nbuf[r, 0, pl.ds(0, L)]
            a10 = nbuf[r, 0, pl.ds(L, L)]
            a01 = nbuf[r, 1, pl.ds(0, L)]
            a11 = nbuf[r, 1, pl.ds(L, L)]
            for n in range(2, DEG, 2):
                a00 = a00 + nbuf[r, n, pl.ds(0, L)]
                a10 = a10 + nbuf[r, n, pl.ds(L, L)]
                a01 = a01 + nbuf[r, n + 1, pl.ds(0, L)]
                a11 = a11 + nbuf[r, n + 1, pl.ds(L, L)]
            obuf[r, pl.ds(0, L)] = sbuf[r, pl.ds(0, L)]
            obuf[r, pl.ds(L, L)] = sbuf[r, pl.ds(L, L)]
            obuf[r, pl.ds(2 * L, L)] = (a00 + a01) * inv
            obuf[r, pl.ds(3 * L, L)] = (a10 + a11) * inv
            return 0
        lax.fori_loop(0, CHUNK, row, 0)

        pltpu.sync_copy(obuf, out.at[pl.ds(row0, CHUNK)])
        return 0

    lax.fori_loop(0, NCHUNK, chunk_body, 0)


@functools.lru_cache(maxsize=1)
def _make_gather_mean():
    return pl.kernel(
        _sc_side_body,
        out_type=jax.ShapeDtypeStruct((B, 2 * ED), jnp.float32),
        mesh=plsc.VectorSubcoreMesh(core_axis_name="c", subcore_axis_name="s",
                                    num_cores=NC, num_subcores=NS),
        scratch_types=[
            pltpu.VMEM((CHUNK,), jnp.int32),
            pltpu.VMEM((CHUNK, DEG), jnp.int32),
            pltpu.VMEM((CHUNK, DEG, ED), jnp.float32),
            pltpu.VMEM((CHUNK, ED), jnp.float32),
            pltpu.VMEM((CHUNK, 2 * ED), jnp.float32),
            pltpu.SemaphoreType.DMA,
            pltpu.SemaphoreType.DMA,
        ],
        compiler_params=pltpu.CompilerParams(use_tc_tiling_on_sc=False),
    )


TW = 16384
NT = (NV + TW - 1) // TW
NVP = NT * TW
SH = (TW // 4).bit_length() - 1


def _tr_body(src_ref, dst_ref):
    d_iota = lax.broadcasted_iota(jnp.int32, (ED, 128), 0)
    l_iota = lax.broadcasted_iota(jnp.int32, (ED, 128), 1)

    def compute(tail):
        ntail = NV - (NT - 1) * TW
        acc = None
        for m in range(4):
            if tail and (TW // 4) * m >= ntail:
                continue
            xc = src_ref[:, (TW // 4) * m:(TW // 4) * (m + 1)]
            if tail:
                col = lax.broadcasted_iota(jnp.int32, (ED, TW // 4), 1)
                xc = jnp.where(col + (TW // 4) * m < ntail, xc, 0.0)
            pm = (l_iota == d_iota + 32 * m).astype(jnp.float32)
            t = lax.dot_general(xc, pm, (((0,), (0,)), ((), ())),
                                preferred_element_type=jnp.float32)
            acc = t if acc is None else acc + t
        return acc

    i = pl.program_id(0)

    @pl.when(i != NT - 1)
    def _():
        dst_ref[...] = compute(False)

    @pl.when(i == NT - 1)
    def _():
        dst_ref[...] = compute(True)


def _linearize_table(t):
    out = pl.pallas_call(
        _tr_body,
        grid=(NT,),
        in_specs=[pl.BlockSpec((ED, TW), lambda i: (0, i))],
        out_specs=pl.BlockSpec((TW // 4, 128), lambda i: (i, 0)),
        out_shape=jax.ShapeDtypeStruct((NVP * ED // 128, 128), jnp.float32),
        compiler_params=pltpu.CompilerParams(fuse_transposed_lhs_in_matmul=True),
    )(t.T)
    return out.reshape(NVP, ED)


def _remap_idx(r):
    return (r & ~(TW - 1)) | ((r & (TW // 4 - 1)) << 2) | ((r >> SH) & 3)


RB = 2048
NB = B // RB


def _tc_body(xu_ref, xv_ref, wut_ref, bu_ref, wvt_ref, bv_ref, out_ref):
    u = jnp.dot(xu_ref[...], wut_ref[...],
                preferred_element_type=jnp.float32) + bu_ref[...]
    v = jnp.dot(xv_ref[...], wvt_ref[...],
                preferred_element_type=jnp.float32) + bv_ref[...]
    out_ref[...] = jnp.sum(u * v, axis=1)[:, None]


def _project_dot(xu, xv, wut, bu2, wvt, bv2):
    out = pl.pallas_call(
        _tc_body,
        grid=(NB,),
        in_specs=[
            pl.BlockSpec((RB, 2 * ED), lambda i: (i, 0)),
            pl.BlockSpec((RB, 2 * ED), lambda i: (i, 0)),
            pl.BlockSpec((2 * ED, ED), lambda i: (0, 0)),
            pl.BlockSpec((1, ED), lambda i: (0, 0)),
            pl.BlockSpec((2 * ED, ED), lambda i: (0, 0)),
            pl.BlockSpec((1, ED), lambda i: (0, 0)),
        ],
        out_specs=pl.BlockSpec((RB, 1), lambda i: (i, 0)),
        out_shape=jax.ShapeDtypeStruct((B, 1), jnp.float32),
    )(xu, xv, wut, bu2, wvt, bv2)
    return out.reshape(B)


def kernel(nodes_u, nodes_v, u_neibs, v_neibs, all_sels, u2e, v2e, Wu, bu, Wv, bv):
    del all_sels
    nodes_u = _remap_idx(nodes_u.astype(jnp.int32))
    nodes_v = _remap_idx(nodes_v.astype(jnp.int32))
    un3 = _remap_idx(u_neibs.astype(jnp.int32)).reshape(NW * NCHUNK, CHUNK, DEG)
    vn3 = _remap_idx(v_neibs.astype(jnp.int32)).reshape(NW * NCHUNK, CHUNK, DEG)
    u2e_lin = _linearize_table(u2e)
    v2e_lin = _linearize_table(v2e)
    gm = _make_gather_mean()
    xu = gm(nodes_u, un3, u2e_lin)
    xv = gm(nodes_v, vn3, v2e_lin)
    return _project_dot(xu, xv, Wu.T, bu[None, :], Wv.T, bv[None, :])
